# Initial kernel scaffold; baseline (speedup 1.0000x reference)
import jax
import jax.numpy as jnp
from jax.experimental import pallas as pl


def kernel(x, edge_index, mask, batch, W0, b0, gamma0, beta0, rm0, rv0, W1, b1, gamma1, beta1, rm1, rv1, W2, b2, gamma2, beta2, rm2, rv2, Wout, bout):
    raise NotImplementedError("write your pallas kernel here")


# speedup vs baseline: 12.2727x; 12.2727x over previous
import functools

import jax
import jax.numpy as jnp
from jax import lax
from jax.experimental import pallas as pl
from jax.experimental.pallas import tpu as pltpu
from jax.experimental.pallas import tpu_sc as plsc

N = 10000
E = 320000
D = 128
H = 128
G = 64
EPS = 1e-5

NC = 2
NS = 16
NW = NC * NS
EPW = E // NW
CH = 128
NFULL = EPW // CH
TAIL = EPW - NFULL * CH
PADN = 10240
RZ = PADN // NS
RSTG = 128

ROWS_TC = 400
GRID_TC = N // ROWS_TC


def _sc_mesh():
    return plsc.VectorSubcoreMesh(core_axis_name="c", subcore_axis_name="s")


@functools.partial(
    pl.kernel,
    out_type=jax.ShapeDtypeStruct((NC * PADN,), jnp.float32),
    mesh=_sc_mesh(),
    scratch_types=[
        pltpu.VMEM((CH,), jnp.int32),
        pltpu.VMEM((TAIL,), jnp.int32),
        pltpu.VMEM((CH,), jnp.float32),
        pltpu.VMEM((TAIL,), jnp.float32),
        pltpu.VMEM((RZ,), jnp.float32),
        pltpu.VMEM_SHARED((PADN,), jnp.float32),
    ],
)
def _deg_kernel(dst_hbm, out_hbm, idx, idx_t, ones, ones_t, stg, deg):
    cid = lax.axis_index("c")
    sid = lax.axis_index("s")
    wid = cid * NS + sid

    def fill0(i, c):
        stg[pl.ds(i * 16, 16)] = jnp.zeros((16,), jnp.float32)
        return c
    lax.fori_loop(0, RZ // 16, fill0, 0)

    def fill1(i, c):
        ones[pl.ds(i * 16, 16)] = jnp.full((16,), 1.0, jnp.float32)
        return c
    lax.fori_loop(0, CH // 16, fill1, 0)
    ones_t[...] = jnp.full((TAIL,), 1.0, jnp.float32)

    pltpu.sync_copy(stg, deg.at[pl.ds(sid * RZ, RZ)])
    plsc.subcore_barrier()

    base = wid * EPW

    def body(c, acc):
        off = pl.multiple_of(base + c * CH, 8)
        pltpu.sync_copy(dst_hbm.at[pl.ds(off, CH)], idx)
        pltpu.sync_copy(ones, deg.at[idx], add=True)
        return acc
    lax.fori_loop(0, NFULL, body, 0)

    off_t = pl.multiple_of(base + NFULL * CH, 8)
    pltpu.sync_copy(dst_hbm.at[pl.ds(off_t, TAIL)], idx_t)
    pltpu.sync_copy(ones_t, deg.at[idx_t], add=True)

    plsc.subcore_barrier()

    pltpu.sync_copy(deg.at[pl.ds(sid * RZ, RZ)], stg)
    pltpu.sync_copy(stg, out_hbm.at[pl.ds(cid * PADN + sid * RZ, RZ)])


@functools.partial(
    pl.kernel,
    out_type=jax.ShapeDtypeStruct((NC, PADN, H), jnp.float32),
    mesh=_sc_mesh(),
    scratch_types=[
        pltpu.VMEM((CH,), jnp.int32),
        pltpu.VMEM((CH,), jnp.int32),
        pltpu.VMEM((CH, H), jnp.float32),
        pltpu.VMEM((TAIL,), jnp.int32),
        pltpu.VMEM((TAIL,), jnp.int32),
        pltpu.VMEM((TAIL, H), jnp.float32),
        pltpu.VMEM((RSTG, H), jnp.float32),
        pltpu.VMEM_SHARED((PADN, H), jnp.float32),
        pltpu.SemaphoreType.DMA,
    ],
)
def _edge_kernel(s_hbm, src_hbm, dst_hbm, out_hbm,
                 sidx, didx, rows, sidx_t, didx_t, rows_t, stg, agg, sem):
    cid = lax.axis_index("c")
    sid = lax.axis_index("s")
    wid = cid * NS + sid
    r0 = sid * RZ

    def fill0(i, c):
        stg[i // 8, pl.ds((i % 8) * 16, 16)] = jnp.zeros((16,), jnp.float32)
        return c
    lax.fori_loop(0, RSTG * (H // 16), fill0, 0)
    for k in range(RZ // RSTG):
        pltpu.sync_copy(stg, agg.at[pl.ds(r0 + k * RSTG, RSTG)])
    plsc.subcore_barrier()

    base = wid * EPW

    def body(c, acc):
        off = pl.multiple_of(base + c * CH, 8)
        pltpu.sync_copy(src_hbm.at[pl.ds(off, CH)], sidx)
        pltpu.sync_copy(dst_hbm.at[pl.ds(off, CH)], didx)
        pltpu.async_copy(s_hbm.at[sidx], rows, sem).wait()
        pltpu.sync_copy(rows, agg.at[didx], add=True)
        return acc
    lax.fori_loop(0, NFULL, body, 0)

    off_t = pl.multiple_of(base + NFULL * CH, 8)
    pltpu.sync_copy(src_hbm.at[pl.ds(off_t, TAIL)], sidx_t)
    pltpu.sync_copy(dst_hbm.at[pl.ds(off_t, TAIL)], didx_t)
    pltpu.async_copy(s_hbm.at[sidx_t], rows_t, sem).wait()
    pltpu.sync_copy(rows_t, agg.at[didx_t], add=True)

    plsc.subcore_barrier()
    for k in range(RZ // RSTG):
        pltpu.sync_copy(agg.at[pl.ds(r0 + k * RSTG, RSTG)], stg)
        pltpu.sync_copy(stg, out_hbm.at[cid, pl.ds(r0 + k * RSTG, RSTG)])


def _prep_body(x_ref, m_ref, d0_ref, d1_ref, w_ref, s_ref, dinv_ref):
    deg = d0_ref[...] + d1_ref[...] + 1.0
    dv = lax.rsqrt(deg)
    dinv_ref[...] = dv
    xm = x_ref[...] * m_ref[...] * dv
    s_ref[...] = jnp.dot(xm, w_ref[...], preferred_element_type=jnp.float32)


def _prep(x, mask, d0, d1, w):
    return pl.pallas_call(
        _prep_body,
        grid=(GRID_TC,),
        in_specs=[
            pl.BlockSpec((ROWS_TC, D), lambda i: (i, 0)),
            pl.BlockSpec((ROWS_TC, 1), lambda i: (i, 0)),
            pl.BlockSpec((ROWS_TC, 1), lambda i: (i, 0)),
            pl.BlockSpec((ROWS_TC, 1), lambda i: (i, 0)),
            pl.BlockSpec((D, H), lambda i: (0, 0)),
        ],
        out_specs=[
            pl.BlockSpec((ROWS_TC, H), lambda i: (i, 0)),
            pl.BlockSpec((ROWS_TC, 1), lambda i: (i, 0)),
        ],
        out_shape=[
            jax.ShapeDtypeStruct((N, H), jnp.float32),
            jax.ShapeDtypeStruct((N, 1), jnp.float32),
        ],
    )(x, mask, d0, d1, w)


def _dense_body(a0_ref, a1_ref, s_ref, dinv_ref, m_ref, b_ref, w_ref, o_ref):
    dv = dinv_ref[...]
    pre = dv * (a0_ref[...] + a1_ref[...] + s_ref[...]) + b_ref[...]
    h = jnp.maximum(pre, 0.0) * m_ref[...] * dv
    o_ref[...] = jnp.dot(h, w_ref[...], preferred_element_type=jnp.float32)


def _dense(a0, a1, s, dinv, mask, b, w):
    return pl.pallas_call(
        _dense_body,
        grid=(GRID_TC,),
        in_specs=[
            pl.BlockSpec((ROWS_TC, H), lambda i: (i, 0)),
            pl.BlockSpec((ROWS_TC, H), lambda i: (i, 0)),
            pl.BlockSpec((ROWS_TC, H), lambda i: (i, 0)),
            pl.BlockSpec((ROWS_TC, 1), lambda i: (i, 0)),
            pl.BlockSpec((ROWS_TC, 1), lambda i: (i, 0)),
            pl.BlockSpec((1, H), lambda i: (0, 0)),
            pl.BlockSpec((H, H), lambda i: (0, 0)),
        ],
        out_specs=pl.BlockSpec((ROWS_TC, H), lambda i: (i, 0)),
        out_shape=jax.ShapeDtypeStruct((N, H), jnp.float32),
    )(a0, a1, s, dinv, mask, b, w)


def _final_body(a0_ref, a1_ref, s_ref, dinv_ref, m_ref, b_ref, batch_ref,
                wout_ref, bout_ref, out_ref, gsum, gcnt):
    i = pl.program_id(0)
    dv = dinv_ref[...]
    pre = dv * (a0_ref[...] + a1_ref[...] + s_ref[...]) + b_ref[...]
    h = jnp.maximum(pre, 0.0) * m_ref[...]
    onehot = (batch_ref[...] ==
              lax.broadcasted_iota(jnp.int32, (ROWS_TC, G), 1)).astype(jnp.float32)
    dn = (((0,), (0,)), ((), ()))
    gs = lax.dot_general(onehot, h, dn, preferred_element_type=jnp.float32)
    cn = lax.dot_general(onehot, jnp.ones((ROWS_TC, H), jnp.float32), dn,
                         preferred_element_type=jnp.float32)

    @pl.when(i == 0)
    def _():
        gsum[...] = gs
        gcnt[...] = cn

    @pl.when(i > 0)
    def _():
        gsum[...] += gs
        gcnt[...] += cn

    @pl.when(i == pl.num_programs(0) - 1)
    def _():
        gr = gsum[...] / jnp.maximum(gcnt[...], 1.0)
        out_ref[...] = (jnp.dot(gr, wout_ref[...],
                                preferred_element_type=jnp.float32)
                        + bout_ref[...])


def _final(a0, a1, s, dinv, mask, b, batch2d, wout, bout2d):
    return pl.pallas_call(
        _final_body,
        grid=(GRID_TC,),
        in_specs=[
            pl.BlockSpec((ROWS_TC, H), lambda i: (i, 0)),
            pl.BlockSpec((ROWS_TC, H), lambda i: (i, 0)),
            pl.BlockSpec((ROWS_TC, H), lambda i: (i, 0)),
            pl.BlockSpec((ROWS_TC, 1), lambda i: (i, 0)),
            pl.BlockSpec((ROWS_TC, 1), lambda i: (i, 0)),
            pl.BlockSpec((1, H), lambda i: (0, 0)),
            pl.BlockSpec((ROWS_TC, 1), lambda i: (i, 0)),
            pl.BlockSpec((H, 1), lambda i: (0, 0)),
            pl.BlockSpec((1, 1), lambda i: (0, 0)),
        ],
        out_specs=pl.BlockSpec((G, 1), lambda i: (0, 0)),
        out_shape=jax.ShapeDtypeStruct((G, 1), jnp.float32),
        scratch_shapes=[
            pltpu.VMEM((G, H), jnp.float32),
            pltpu.VMEM((G, H), jnp.float32),
        ],
        compiler_params=pltpu.CompilerParams(
            dimension_semantics=("arbitrary",)),
    )(a0, a1, s, dinv, mask, b, batch2d, wout, bout2d)


def kernel(x, edge_index, mask, batch,
           W0, b0, gamma0, beta0, rm0, rv0,
           W1, b1, gamma1, beta1, rm1, rv1,
           W2, b2, gamma2, beta2, rm2, rv2,
           Wout, bout):
    src = edge_index[0]
    dst = edge_index[1]

    def fold(Wl, bl, gl, bel, rml, rvl):
        scale = gl * lax.rsqrt(rvl + EPS)
        return Wl * scale[None, :], (bl * scale + bel - rml * scale).reshape(1, H)

    W0p, b0p = fold(W0, b0, gamma0, beta0, rm0, rv0)
    W1p, b1p = fold(W1, b1, gamma1, beta1, rm1, rv1)
    W2p, b2p = fold(W2, b2, gamma2, beta2, rm2, rv2)

    degp = _deg_kernel(dst).reshape(NC, PADN)
    d0 = degp[0, :N].reshape(N, 1)
    d1 = degp[1, :N].reshape(N, 1)

    s1, dinv = _prep(x, mask, d0, d1, W0p)

    agg = _edge_kernel(s1, src, dst)
    s2 = _dense(agg[0, :N], agg[1, :N], s1, dinv, mask, b0p, W1p)
    agg = _edge_kernel(s2, src, dst)
    s3 = _dense(agg[0, :N], agg[1, :N], s2, dinv, mask, b1p, W2p)
    agg = _edge_kernel(s3, src, dst)
    return _final(agg[0, :N], agg[1, :N], s3, dinv, mask, b2p,
                  batch.reshape(N, 1), Wout, bout.reshape(1, 1))
